# baseline (speedup 1.0000x reference)
import jax
import jax.numpy as jnp
from jax import lax
from jax.experimental import pallas as pl
from jax.experimental.pallas import tpu as pltpu, tpu_sc as plsc

import functools


def kernel(x, tables):
    F, V, D = tables.shape
    B, F2 = x.shape
    assert F == F2

    info = plsc.get_sparse_core_info()
    NC, NS, L = info.num_cores, info.num_subcores, info.num_lanes
    NW = NC * NS
    N = B * F
    per_w = N // NW
    assert per_w * NW == N and per_w % F == 0 and per_w % L == 0
    GCH = 128
    n_g = per_w // GCH
    assert n_g * GCH == per_w

    flat_tables = tables.reshape(F * V, D)
    x_flat = x.reshape(N).astype(jnp.int32)

    mesh = plsc.VectorSubcoreMesh(core_axis_name="c", subcore_axis_name="s")

    @functools.partial(
        pl.kernel,
        mesh=mesh,
        compiler_params=pltpu.CompilerParams(use_tc_tiling_on_sc=False),
        out_type=jax.ShapeDtypeStruct((N, D), jnp.float32),
        scratch_types=[
            pltpu.VMEM((per_w,), jnp.int32),
            pltpu.VMEM((per_w, D), jnp.float32),
            pltpu.SemaphoreType.DMA,
            pltpu.SemaphoreType.DMA,
        ],
    )
    def emb_kernel(tab_hbm, idx_hbm, out_hbm, idx_v, rows_v, sem_i, sem_g):
        wid = lax.axis_index("s") * NC + lax.axis_index("c")
        base = wid * per_w

        pltpu.async_copy(idx_hbm.at[pl.ds(base, per_w)], idx_v, sem_i).wait()

        def add_offs(i, _):
            s = pl.ds(i * L, L)
            f = lax.rem(i * L + lax.iota(jnp.int32, L), F)
            idx_v[s] = idx_v[s] + f * V
            return 0

        lax.fori_loop(0, per_w // L, add_offs, 0)

        copies = [
            pltpu.make_async_copy(
                tab_hbm.at[idx_v.at[pl.ds(j * GCH, GCH)]],
                rows_v.at[pl.ds(j * GCH, GCH)],
                sem_g,
            )
            for j in range(n_g)
        ]
        for c in copies:
            c.start()
        for c in copies:
            c.wait()

        pltpu.async_copy(rows_v, out_hbm.at[pl.ds(base, per_w)], sem_i).wait()

    out_flat = emb_kernel(flat_tables, x_flat)
    return out_flat.reshape(B, F, D)
